# baseline (speedup 1.0000x reference)
import functools

import jax
import jax.numpy as jnp
from jax import lax
from jax.experimental import pallas as pl
from jax.experimental.pallas import tpu as pltpu
from jax.experimental.pallas import tpu_sc as plsc

_B = 16384
_NNEG = 20
_D = 16

_info = plsc.get_sparse_core_info()
_NC, _NS = _info.num_cores, _info.num_subcores
_NW = _NC * _NS

_CHUNK = 512
_U_PER_W = _B // _NW
_N_PER_W = _B * _NNEG // _NW
_N_CH = _N_PER_W // _CHUNK
_NBUF = 4
_LEAD = 3


def _body(u_hbm, p_hbm, n_hbm, utab, itab, uo, po, no,
          idx_u, idx_p, idx_n, u_rows, p_rows, ring, gsem, wsem, usem, psem):
    wid = lax.axis_index("s") * _NC + lax.axis_index("c")
    ub = wid * _U_PER_W
    nb = wid * _N_PER_W

    pltpu.sync_copy(u_hbm.at[wid], idx_u)
    pltpu.sync_copy(p_hbm.at[wid], idx_p)
    pltpu.async_copy(utab.at[idx_u], u_rows, usem)
    pltpu.async_copy(itab.at[idx_p], p_rows, psem)
    pltpu.sync_copy(n_hbm.at[wid], idx_n)

    def g_start(c, b):
        pltpu.async_copy(itab.at[idx_n.at[c]], ring.at[b], gsem.at[b])

    def g_wait(b):
        pltpu.make_async_copy(itab.at[idx_n.at[0]], ring.at[b], gsem.at[b]).wait()

    def w_start(c, b):
        pltpu.async_copy(ring.at[b], no.at[pl.ds(nb + c * _CHUNK, _CHUNK)], wsem.at[b])

    def w_wait(b):
        pltpu.make_async_copy(ring.at[b], no.at[pl.ds(nb, _CHUNK)], wsem.at[b]).wait()

    for c in range(_LEAD):
        g_start(c, c)

    for b in range(_NBUF):
        g_wait(b)
        w_start(b, b)
        if b >= 1:
            w_wait((b + _LEAD) % _NBUF)
        g_start(b + _LEAD, (b + _LEAD) % _NBUF)

    def mid(jo, carry):
        for b in range(_NBUF):
            j = jo * _NBUF + b
            g_wait(b)
            w_start(j, b)
            w_wait((b + _LEAD) % _NBUF)
            g_start(j + _LEAD, (b + _LEAD) % _NBUF)
        return carry
    lax.fori_loop(1, _N_CH // _NBUF - 1, mid, 0)

    for b in range(_NBUF):
        j = _N_CH - _NBUF + b
        g_wait(b)
        w_start(j, b)
        if b == 0:
            w_wait(_NBUF - 1)
            g_start(_N_CH - 1, _NBUF - 1)
    for b in range(_NBUF):
        w_wait(b)

    pltpu.make_async_copy(utab.at[idx_u], u_rows, usem).wait()
    pltpu.async_copy(u_rows, uo.at[pl.ds(ub, _U_PER_W)], usem)
    pltpu.make_async_copy(itab.at[idx_p], p_rows, psem).wait()
    pltpu.async_copy(p_rows, po.at[pl.ds(ub, _U_PER_W)], psem)
    pltpu.make_async_copy(u_rows, uo.at[pl.ds(ub, _U_PER_W)], usem).wait()
    pltpu.make_async_copy(p_rows, po.at[pl.ds(ub, _U_PER_W)], psem).wait()


_gather = functools.partial(
    pl.kernel,
    mesh=plsc.VectorSubcoreMesh(core_axis_name="c", subcore_axis_name="s"),
    compiler_params=pltpu.CompilerParams(use_tc_tiling_on_sc=False),
    out_type=[
        jax.ShapeDtypeStruct((_B, _D), jnp.float32),
        jax.ShapeDtypeStruct((_B, _D), jnp.float32),
        jax.ShapeDtypeStruct((_B * _NNEG, _D), jnp.float32),
    ],
    scratch_types=[
        pltpu.VMEM((_U_PER_W,), jnp.int32),
        pltpu.VMEM((_U_PER_W,), jnp.int32),
        pltpu.VMEM((_N_CH, _CHUNK), jnp.int32),
        pltpu.VMEM((_U_PER_W, _D), jnp.float32),
        pltpu.VMEM((_U_PER_W, _D), jnp.float32),
        pltpu.VMEM((_NBUF, _CHUNK, _D), jnp.float32),
        pltpu.SemaphoreType.DMA((_NBUF,)),
        pltpu.SemaphoreType.DMA((_NBUF,)),
        pltpu.SemaphoreType.DMA,
        pltpu.SemaphoreType.DMA,
    ],
)(_body)


_NV = 1000000
_RT = _NV // 8
_CCH = 512
_NCH_FULL = _NV // _CCH
_REM = _NV - _NCH_FULL * _CCH


def _detile_body(t_hbm, tail_hbm, o_hbm, vin, vtail, vout):
    wid = lax.axis_index("s") * _NC + lax.axis_index("c")
    nch = jnp.where(wid < _NCH_FULL % _NW, _NCH_FULL // _NW + 1, _NCH_FULL // _NW)
    dlanes = lax.iota(jnp.int32, 16)

    def transpose_cols(src, ncols):
        def jo_body(jo, carry):
            for q in range(8):
                col = jo * 8 + q
                v = plsc.load_gather(src, [dlanes, jnp.zeros((16,), jnp.int32) + col])
                vout[jo, pl.ds(16 * q, 16)] = v
            return carry
        lax.fori_loop(0, ncols // 8, jo_body, 0)

    def k_body(k, carry):
        c = wid + k * _NW
        pltpu.sync_copy(t_hbm.at[:, pl.ds(c * _CCH, _CCH)], vin)
        transpose_cols(vin, _CCH)
        pltpu.sync_copy(vout, o_hbm.at[pl.ds(c * (_CCH // 8), _CCH // 8)])
        return carry
    lax.fori_loop(0, nch, k_body, 0)

    @pl.when(wid == _NW - 1)
    def _():
        pltpu.sync_copy(tail_hbm, vtail)
        transpose_cols(vtail, _REM)
        pltpu.sync_copy(vout.at[pl.ds(0, _REM // 8)],
                        o_hbm.at[pl.ds(_NCH_FULL * (_CCH // 8), _REM // 8)])


_detile = functools.partial(
    pl.kernel,
    mesh=plsc.VectorSubcoreMesh(core_axis_name="c", subcore_axis_name="s"),
    compiler_params=pltpu.CompilerParams(needs_layout_passes=False,
                                         disable_bounds_checks=True),
    out_type=jax.ShapeDtypeStruct((_RT, 128), jnp.float32),
    scratch_types=[
        pltpu.VMEM((_D, _CCH), jnp.float32),
        pltpu.VMEM((_D, 128), jnp.float32),
        pltpu.VMEM((_CCH // 8, 128), jnp.float32),
    ],
)(_detile_body)


def _to_linear(t):
    tail = jnp.pad(t[_NCH_FULL * _CCH:], ((0, 128 - _REM), (0, 0)))
    return _detile(t.T, tail.T).reshape(_NV, _D)


def kernel(users, pos_items, neg_items, user_embedding, item_embedding):
    ut_lin = _to_linear(user_embedding)
    it_lin = _to_linear(item_embedding)
    u = users.reshape(_NW, _U_PER_W)
    p = pos_items.reshape(_NW, _U_PER_W)
    n = neg_items.reshape(_NW, _N_CH, _CHUNK)
    ue, pe, ne = _gather(u, p, n, ut_lin, it_lin)
    return (ue.reshape(_B, 1, _D), pe.reshape(_B, 1, _D),
            ne.reshape(_B, _NNEG, _D))


# speedup vs baseline: 1.2740x; 1.2740x over previous
import functools

import jax
import jax.numpy as jnp
from jax import lax
from jax.experimental import pallas as pl
from jax.experimental.pallas import tpu as pltpu
from jax.experimental.pallas import tpu_sc as plsc

_B = 16384
_NNEG = 20
_D = 16

_info = plsc.get_sparse_core_info()
_NC, _NS = _info.num_cores, _info.num_subcores
_NW = _NC * _NS

_CHUNK = 512
_U_PER_W = _B // _NW
_N_PER_W = _B * _NNEG // _NW
_N_CH = _N_PER_W // _CHUNK
_NBUF = 4
_LEAD = 3


def _body(u_hbm, p_hbm, n_hbm, utab, itab, uo, po, no,
          idx_u, idx_p, idx_n, u_rows, p_rows, ring, gsem, wsem, usem, psem):
    wid = lax.axis_index("s") * _NC + lax.axis_index("c")
    ub = wid * _U_PER_W
    nb = wid * _N_PER_W

    pltpu.sync_copy(u_hbm.at[wid], idx_u)
    pltpu.sync_copy(p_hbm.at[wid], idx_p)
    pltpu.async_copy(utab.at[idx_u], u_rows, usem)
    pltpu.async_copy(itab.at[idx_p], p_rows, psem)
    pltpu.sync_copy(n_hbm.at[wid], idx_n)

    def g_start(c, b):
        pltpu.async_copy(itab.at[idx_n.at[c]], ring.at[b], gsem.at[b])

    def g_wait(b):
        pltpu.make_async_copy(itab.at[idx_n.at[0]], ring.at[b], gsem.at[b]).wait()

    def w_start(c, b):
        pltpu.async_copy(ring.at[b], no.at[pl.ds(nb + c * _CHUNK, _CHUNK)], wsem.at[b])

    def w_wait(b):
        pltpu.make_async_copy(ring.at[b], no.at[pl.ds(nb, _CHUNK)], wsem.at[b]).wait()

    for c in range(_LEAD):
        g_start(c, c)

    for b in range(_NBUF):
        g_wait(b)
        w_start(b, b)
        if b >= 1:
            w_wait((b + _LEAD) % _NBUF)
        g_start(b + _LEAD, (b + _LEAD) % _NBUF)

    def mid(jo, carry):
        for b in range(_NBUF):
            j = jo * _NBUF + b
            g_wait(b)
            w_start(j, b)
            w_wait((b + _LEAD) % _NBUF)
            g_start(j + _LEAD, (b + _LEAD) % _NBUF)
        return carry
    lax.fori_loop(1, _N_CH // _NBUF - 1, mid, 0)

    for b in range(_NBUF):
        j = _N_CH - _NBUF + b
        g_wait(b)
        w_start(j, b)
        if b == 0:
            w_wait(_NBUF - 1)
            g_start(_N_CH - 1, _NBUF - 1)
    for b in range(_NBUF):
        w_wait(b)

    pltpu.make_async_copy(utab.at[idx_u], u_rows, usem).wait()
    pltpu.async_copy(u_rows, uo.at[pl.ds(ub, _U_PER_W)], usem)
    pltpu.make_async_copy(itab.at[idx_p], p_rows, psem).wait()
    pltpu.async_copy(p_rows, po.at[pl.ds(ub, _U_PER_W)], psem)
    pltpu.make_async_copy(u_rows, uo.at[pl.ds(ub, _U_PER_W)], usem).wait()
    pltpu.make_async_copy(p_rows, po.at[pl.ds(ub, _U_PER_W)], psem).wait()


_gather = functools.partial(
    pl.kernel,
    mesh=plsc.VectorSubcoreMesh(core_axis_name="c", subcore_axis_name="s"),
    compiler_params=pltpu.CompilerParams(use_tc_tiling_on_sc=False),
    out_type=[
        jax.ShapeDtypeStruct((_B, _D), jnp.float32),
        jax.ShapeDtypeStruct((_B, _D), jnp.float32),
        jax.ShapeDtypeStruct((_B * _NNEG, _D), jnp.float32),
    ],
    scratch_types=[
        pltpu.VMEM((_U_PER_W,), jnp.int32),
        pltpu.VMEM((_U_PER_W,), jnp.int32),
        pltpu.VMEM((_N_CH, _CHUNK), jnp.int32),
        pltpu.VMEM((_U_PER_W, _D), jnp.float32),
        pltpu.VMEM((_U_PER_W, _D), jnp.float32),
        pltpu.VMEM((_NBUF, _CHUNK, _D), jnp.float32),
        pltpu.SemaphoreType.DMA((_NBUF,)),
        pltpu.SemaphoreType.DMA((_NBUF,)),
        pltpu.SemaphoreType.DMA,
        pltpu.SemaphoreType.DMA,
    ],
)(_body)


_NV = 1000000
_RT = _NV // 8
_CCH = 512
_NCH_FULL = _NV // _CCH
_REM = _NV - _NCH_FULL * _CCH


def _detile_body(t_hbm, tail_hbm, o_hbm, vin, vtail, vout):
    wid = lax.axis_index("s") * _NC + lax.axis_index("c")
    nch = jnp.where(wid < _NCH_FULL % _NW, _NCH_FULL // _NW + 1, _NCH_FULL // _NW)
    dlanes = lax.iota(jnp.int32, 16)

    def transpose_cols(src, ncols):
        @plsc.parallel_loop(0, ncols, unroll=8)
        def _(col):
            v = plsc.load_gather(src, [dlanes, jnp.zeros((16,), jnp.int32) + col])
            vout[col // 8, pl.ds(16 * (col % 8), 16)] = v

    def k_body(k, carry):
        c = wid + k * _NW
        pltpu.sync_copy(t_hbm.at[:, pl.ds(c * _CCH, _CCH)], vin)
        transpose_cols(vin, _CCH)
        pltpu.sync_copy(vout, o_hbm.at[pl.ds(c * (_CCH // 8), _CCH // 8)])
        return carry
    lax.fori_loop(0, nch, k_body, 0)

    @pl.when(wid == _NW - 1)
    def _():
        pltpu.sync_copy(tail_hbm, vtail)
        transpose_cols(vtail, _REM)
        pltpu.sync_copy(vout.at[pl.ds(0, _REM // 8)],
                        o_hbm.at[pl.ds(_NCH_FULL * (_CCH // 8), _REM // 8)])


_detile = functools.partial(
    pl.kernel,
    mesh=plsc.VectorSubcoreMesh(core_axis_name="c", subcore_axis_name="s"),
    compiler_params=pltpu.CompilerParams(needs_layout_passes=False,
                                         disable_bounds_checks=True),
    out_type=jax.ShapeDtypeStruct((_RT, 128), jnp.float32),
    scratch_types=[
        pltpu.VMEM((_D, _CCH), jnp.float32),
        pltpu.VMEM((_D, 128), jnp.float32),
        pltpu.VMEM((_CCH // 8, 128), jnp.float32),
    ],
)(_detile_body)


def _to_linear(t):
    tail = jnp.pad(t[_NCH_FULL * _CCH:], ((0, 128 - _REM), (0, 0)))
    return _detile(t.T, tail.T).reshape(_NV, _D)


def kernel(users, pos_items, neg_items, user_embedding, item_embedding):
    ut_lin = _to_linear(user_embedding)
    it_lin = _to_linear(item_embedding)
    u = users.reshape(_NW, _U_PER_W)
    p = pos_items.reshape(_NW, _U_PER_W)
    n = neg_items.reshape(_NW, _N_CH, _CHUNK)
    ue, pe, ne = _gather(u, p, n, ut_lin, it_lin)
    return (ue.reshape(_B, 1, _D), pe.reshape(_B, 1, _D),
            ne.reshape(_B, _NNEG, _D))


# speedup vs baseline: 1.2962x; 1.0174x over previous
import functools

import jax
import jax.numpy as jnp
from jax import lax
from jax.experimental import pallas as pl
from jax.experimental.pallas import tpu as pltpu
from jax.experimental.pallas import tpu_sc as plsc

_B = 16384
_NNEG = 20
_D = 16

_info = plsc.get_sparse_core_info()
_NC, _NS = _info.num_cores, _info.num_subcores
_NW = _NC * _NS

_CHUNK = 512
_U_PER_W = _B // _NW
_N_PER_W = _B * _NNEG // _NW
_N_CH = _N_PER_W // _CHUNK
_NBUF = 4
_LEAD = 3


def _body(u_hbm, p_hbm, n_hbm, utab, itab, uo, po, no,
          idx_u, idx_p, idx_n, u_rows, p_rows, ring, gsem, wsem, usem, psem):
    wid = lax.axis_index("s") * _NC + lax.axis_index("c")
    ub = wid * _U_PER_W
    nb = wid * _N_PER_W

    pltpu.sync_copy(u_hbm.at[wid], idx_u)
    pltpu.sync_copy(p_hbm.at[wid], idx_p)
    pltpu.async_copy(utab.at[idx_u], u_rows, usem)
    pltpu.async_copy(itab.at[idx_p], p_rows, psem)
    pltpu.sync_copy(n_hbm.at[wid], idx_n)

    def g_start(c, b):
        pltpu.async_copy(itab.at[idx_n.at[c]], ring.at[b], gsem.at[b])

    def g_wait(b):
        pltpu.make_async_copy(itab.at[idx_n.at[0]], ring.at[b], gsem.at[b]).wait()

    def w_start(c, b):
        pltpu.async_copy(ring.at[b], no.at[pl.ds(nb + c * _CHUNK, _CHUNK)], wsem.at[b])

    def w_wait(b):
        pltpu.make_async_copy(ring.at[b], no.at[pl.ds(nb, _CHUNK)], wsem.at[b]).wait()

    for c in range(_LEAD):
        g_start(c, c)

    for b in range(_NBUF):
        g_wait(b)
        w_start(b, b)
        if b >= 1:
            w_wait((b + _LEAD) % _NBUF)
        g_start(b + _LEAD, (b + _LEAD) % _NBUF)

    def mid(jo, carry):
        for b in range(_NBUF):
            j = jo * _NBUF + b
            g_wait(b)
            w_start(j, b)
            w_wait((b + _LEAD) % _NBUF)
            g_start(j + _LEAD, (b + _LEAD) % _NBUF)
        return carry
    lax.fori_loop(1, _N_CH // _NBUF - 1, mid, 0)

    for b in range(_NBUF):
        j = _N_CH - _NBUF + b
        g_wait(b)
        w_start(j, b)
        if b == 0:
            w_wait(_NBUF - 1)
            g_start(_N_CH - 1, _NBUF - 1)
    for b in range(_NBUF):
        w_wait(b)

    pltpu.make_async_copy(utab.at[idx_u], u_rows, usem).wait()
    pltpu.async_copy(u_rows, uo.at[pl.ds(ub, _U_PER_W)], usem)
    pltpu.make_async_copy(itab.at[idx_p], p_rows, psem).wait()
    pltpu.async_copy(p_rows, po.at[pl.ds(ub, _U_PER_W)], psem)
    pltpu.make_async_copy(u_rows, uo.at[pl.ds(ub, _U_PER_W)], usem).wait()
    pltpu.make_async_copy(p_rows, po.at[pl.ds(ub, _U_PER_W)], psem).wait()


_gather = functools.partial(
    pl.kernel,
    mesh=plsc.VectorSubcoreMesh(core_axis_name="c", subcore_axis_name="s"),
    compiler_params=pltpu.CompilerParams(use_tc_tiling_on_sc=False),
    out_type=[
        jax.ShapeDtypeStruct((_B, _D), jnp.float32),
        jax.ShapeDtypeStruct((_B, _D), jnp.float32),
        jax.ShapeDtypeStruct((_B * _NNEG, _D), jnp.float32),
    ],
    scratch_types=[
        pltpu.VMEM((_U_PER_W,), jnp.int32),
        pltpu.VMEM((_U_PER_W,), jnp.int32),
        pltpu.VMEM((_N_CH, _CHUNK), jnp.int32),
        pltpu.VMEM((_U_PER_W, _D), jnp.float32),
        pltpu.VMEM((_U_PER_W, _D), jnp.float32),
        pltpu.VMEM((_NBUF, _CHUNK, _D), jnp.float32),
        pltpu.SemaphoreType.DMA((_NBUF,)),
        pltpu.SemaphoreType.DMA((_NBUF,)),
        pltpu.SemaphoreType.DMA,
        pltpu.SemaphoreType.DMA,
    ],
)(_body)


_NV = 1000000
_RT = _NV // 8
_CCH = 512
_NCH_FULL = _NV // _CCH
_REM = _NV - _NCH_FULL * _CCH


def _detile_body(t_hbm, tail_hbm, o_hbm, vin, vtail, vout):
    wid = lax.axis_index("s") * _NC + lax.axis_index("c")
    nch = jnp.where(wid < _NCH_FULL % _NW, _NCH_FULL // _NW + 1, _NCH_FULL // _NW)
    dlanes = lax.iota(jnp.int32, 16)

    def transpose_cols(src, ncols):
        @plsc.parallel_loop(0, ncols, unroll=16)
        def _(col):
            v = plsc.load_gather(src, [dlanes, jnp.zeros((16,), jnp.int32) + col])
            vout[col // 8, pl.ds(16 * (col % 8), 16)] = v

    def k_body(k, carry):
        c = wid + k * _NW
        pltpu.sync_copy(t_hbm.at[:, pl.ds(c * _CCH, _CCH)], vin)
        transpose_cols(vin, _CCH)
        pltpu.sync_copy(vout, o_hbm.at[pl.ds(c * (_CCH // 8), _CCH // 8)])
        return carry
    lax.fori_loop(0, nch, k_body, 0)

    @pl.when(wid == _NW - 1)
    def _():
        pltpu.sync_copy(tail_hbm, vtail)
        transpose_cols(vtail, _REM)
        pltpu.sync_copy(vout.at[pl.ds(0, _REM // 8)],
                        o_hbm.at[pl.ds(_NCH_FULL * (_CCH // 8), _REM // 8)])


_detile = functools.partial(
    pl.kernel,
    mesh=plsc.VectorSubcoreMesh(core_axis_name="c", subcore_axis_name="s"),
    compiler_params=pltpu.CompilerParams(needs_layout_passes=False,
                                         disable_bounds_checks=True),
    out_type=jax.ShapeDtypeStruct((_RT, 128), jnp.float32),
    scratch_types=[
        pltpu.VMEM((_D, _CCH), jnp.float32),
        pltpu.VMEM((_D, 128), jnp.float32),
        pltpu.VMEM((_CCH // 8, 128), jnp.float32),
    ],
)(_detile_body)


def _to_linear(t):
    tail = jnp.pad(t[_NCH_FULL * _CCH:], ((0, 128 - _REM), (0, 0)))
    return _detile(t.T, tail.T).reshape(_NV, _D)


def kernel(users, pos_items, neg_items, user_embedding, item_embedding):
    ut_lin = _to_linear(user_embedding)
    it_lin = _to_linear(item_embedding)
    u = users.reshape(_NW, _U_PER_W)
    p = pos_items.reshape(_NW, _U_PER_W)
    n = neg_items.reshape(_NW, _N_CH, _CHUNK)
    ue, pe, ne = _gather(u, p, n, ut_lin, it_lin)
    return (ue.reshape(_B, 1, _D), pe.reshape(_B, 1, _D),
            ne.reshape(_B, _NNEG, _D))


# speedup vs baseline: 1.3009x; 1.0036x over previous
import functools

import jax
import jax.numpy as jnp
from jax import lax
from jax.experimental import pallas as pl
from jax.experimental.pallas import tpu as pltpu
from jax.experimental.pallas import tpu_sc as plsc

_B = 16384
_NNEG = 20
_D = 16

_info = plsc.get_sparse_core_info()
_NC, _NS = _info.num_cores, _info.num_subcores
_NW = _NC * _NS

_CHUNK = 512
_U_PER_W = _B // _NW
_N_PER_W = _B * _NNEG // _NW
_N_CH = _N_PER_W // _CHUNK
_NBUF = 4
_LEAD = 3


def _body(u_hbm, p_hbm, n_hbm, utab, itab, uo, po, no,
          idx_u, idx_p, idx_n, u_rows, p_rows, ring, gsem, wsem, usem, psem):
    wid = lax.axis_index("s") * _NC + lax.axis_index("c")
    ub = wid * _U_PER_W
    nb = wid * _N_PER_W

    pltpu.sync_copy(u_hbm.at[wid], idx_u)
    pltpu.sync_copy(p_hbm.at[wid], idx_p)
    pltpu.async_copy(utab.at[idx_u], u_rows, usem)
    pltpu.async_copy(itab.at[idx_p], p_rows, psem)
    pltpu.sync_copy(n_hbm.at[wid], idx_n)

    def g_start(c, b):
        pltpu.async_copy(itab.at[idx_n.at[c]], ring.at[b], gsem.at[b])

    def g_wait(b):
        pltpu.make_async_copy(itab.at[idx_n.at[0]], ring.at[b], gsem.at[b]).wait()

    def w_start(c, b):
        pltpu.async_copy(ring.at[b], no.at[pl.ds(nb + c * _CHUNK, _CHUNK)], wsem.at[b])

    def w_wait(b):
        pltpu.make_async_copy(ring.at[b], no.at[pl.ds(nb, _CHUNK)], wsem.at[b]).wait()

    for c in range(_LEAD):
        g_start(c, c)

    for b in range(_NBUF):
        g_wait(b)
        w_start(b, b)
        if b >= 1:
            w_wait((b + _LEAD) % _NBUF)
        g_start(b + _LEAD, (b + _LEAD) % _NBUF)

    def mid(jo, carry):
        for b in range(_NBUF):
            j = jo * _NBUF + b
            g_wait(b)
            w_start(j, b)
            w_wait((b + _LEAD) % _NBUF)
            g_start(j + _LEAD, (b + _LEAD) % _NBUF)
        return carry
    lax.fori_loop(1, _N_CH // _NBUF - 1, mid, 0)

    for b in range(_NBUF):
        j = _N_CH - _NBUF + b
        g_wait(b)
        w_start(j, b)
        if b == 0:
            w_wait(_NBUF - 1)
            g_start(_N_CH - 1, _NBUF - 1)
    for b in range(_NBUF):
        w_wait(b)

    pltpu.make_async_copy(utab.at[idx_u], u_rows, usem).wait()
    pltpu.async_copy(u_rows, uo.at[pl.ds(ub, _U_PER_W)], usem)
    pltpu.make_async_copy(itab.at[idx_p], p_rows, psem).wait()
    pltpu.async_copy(p_rows, po.at[pl.ds(ub, _U_PER_W)], psem)
    pltpu.make_async_copy(u_rows, uo.at[pl.ds(ub, _U_PER_W)], usem).wait()
    pltpu.make_async_copy(p_rows, po.at[pl.ds(ub, _U_PER_W)], psem).wait()


_gather = functools.partial(
    pl.kernel,
    mesh=plsc.VectorSubcoreMesh(core_axis_name="c", subcore_axis_name="s"),
    compiler_params=pltpu.CompilerParams(use_tc_tiling_on_sc=False),
    out_type=[
        jax.ShapeDtypeStruct((_B, _D), jnp.float32),
        jax.ShapeDtypeStruct((_B, _D), jnp.float32),
        jax.ShapeDtypeStruct((_B * _NNEG, _D), jnp.float32),
    ],
    scratch_types=[
        pltpu.VMEM((_U_PER_W,), jnp.int32),
        pltpu.VMEM((_U_PER_W,), jnp.int32),
        pltpu.VMEM((_N_CH, _CHUNK), jnp.int32),
        pltpu.VMEM((_U_PER_W, _D), jnp.float32),
        pltpu.VMEM((_U_PER_W, _D), jnp.float32),
        pltpu.VMEM((_NBUF, _CHUNK, _D), jnp.float32),
        pltpu.SemaphoreType.DMA((_NBUF,)),
        pltpu.SemaphoreType.DMA((_NBUF,)),
        pltpu.SemaphoreType.DMA,
        pltpu.SemaphoreType.DMA,
    ],
)(_body)


_NV = 1000000
_RT = _NV // 8
_CCH = 512
_NCH_FULL = _NV // _CCH
_REM = _NV - _NCH_FULL * _CCH


def _detile_body(t_hbm, tail_hbm, t2_hbm, tail2_hbm, o_hbm, o2_hbm,
                 vin, vtail, vout):
    wid = lax.axis_index("s") * _NC + lax.axis_index("c")
    nch = jnp.where(wid < _NCH_FULL % _NW, _NCH_FULL // _NW + 1, _NCH_FULL // _NW)
    dlanes = lax.iota(jnp.int32, 16)

    def transpose_cols(src, ncols):
        @plsc.parallel_loop(0, ncols, unroll=16)
        def _(col):
            v = plsc.load_gather(src, [dlanes, jnp.zeros((16,), jnp.int32) + col])
            vout[col // 8, pl.ds(16 * (col % 8), 16)] = v

    for t, tl, o in ((t_hbm, tail_hbm, o_hbm), (t2_hbm, tail2_hbm, o2_hbm)):
        def k_body(k, carry, t=t, o=o):
            c = wid + k * _NW
            pltpu.sync_copy(t.at[:, pl.ds(c * _CCH, _CCH)], vin)
            transpose_cols(vin, _CCH)
            pltpu.sync_copy(vout, o.at[pl.ds(c * (_CCH // 8), _CCH // 8)])
            return carry
        lax.fori_loop(0, nch, k_body, 0)

        @pl.when(wid == _NW - 1)
        def _(tl=tl, o=o):
            pltpu.sync_copy(tl, vtail)
            transpose_cols(vtail, _REM)
            pltpu.sync_copy(vout.at[pl.ds(0, _REM // 8)],
                            o.at[pl.ds(_NCH_FULL * (_CCH // 8), _REM // 8)])


_detile = functools.partial(
    pl.kernel,
    mesh=plsc.VectorSubcoreMesh(core_axis_name="c", subcore_axis_name="s"),
    compiler_params=pltpu.CompilerParams(needs_layout_passes=False,
                                         disable_bounds_checks=True),
    out_type=[jax.ShapeDtypeStruct((_RT, 128), jnp.float32),
              jax.ShapeDtypeStruct((_RT, 128), jnp.float32)],
    scratch_types=[
        pltpu.VMEM((_D, _CCH), jnp.float32),
        pltpu.VMEM((_D, 128), jnp.float32),
        pltpu.VMEM((_CCH // 8, 128), jnp.float32),
    ],
)(_detile_body)


def _tail_pad(t):
    return jnp.pad(t[_NCH_FULL * _CCH:], ((0, 128 - _REM), (0, 0))).T


def kernel(users, pos_items, neg_items, user_embedding, item_embedding):
    ut128, it128 = _detile(user_embedding.T, _tail_pad(user_embedding),
                           item_embedding.T, _tail_pad(item_embedding))
    ut_lin = ut128.reshape(_NV, _D)
    it_lin = it128.reshape(_NV, _D)
    u = users.reshape(_NW, _U_PER_W)
    p = pos_items.reshape(_NW, _U_PER_W)
    n = neg_items.reshape(_NW, _N_CH, _CHUNK)
    ue, pe, ne = _gather(u, p, n, ut_lin, it_lin)
    return (ue.reshape(_B, 1, _D), pe.reshape(_B, 1, _D),
            ne.reshape(_B, _NNEG, _D))
